# baseline (speedup 1.0000x reference)
import jax
import jax.numpy as jnp
from jax.experimental import pallas as pl
from jax.experimental.pallas import tpu as pltpu

N_NODES = 10000
D = 128
ROW_BLK = 1000


def _dense_body(x_ref, mean_ref, ws_ref, wn_ref, h_ref):
    acc = jnp.dot(x_ref[...], ws_ref[...], preferred_element_type=jnp.float32)
    acc += jnp.dot(mean_ref[...], wn_ref[...], preferred_element_type=jnp.float32)
    h_ref[...] = jnp.maximum(acc, 0.0)


def _dense_stage(x, mean_agg, W_self, W_neigh):
    grid = (N_NODES // ROW_BLK,)
    return pl.pallas_call(
        _dense_body,
        out_shape=jax.ShapeDtypeStruct((N_NODES, D), jnp.float32),
        grid=grid,
        in_specs=[
            pl.BlockSpec((ROW_BLK, D), lambda i: (i, 0)),
            pl.BlockSpec((ROW_BLK, D), lambda i: (i, 0)),
            pl.BlockSpec((D, D), lambda i: (0, 0)),
            pl.BlockSpec((D, D), lambda i: (0, 0)),
        ],
        out_specs=pl.BlockSpec((ROW_BLK, D), lambda i: (i, 0)),
    )(x, mean_agg, W_self, W_neigh)


def kernel(x, edge_index, src, dst, W_self, W_neigh, node_biases, mu):
    src_e = edge_index[0].astype(jnp.int32)
    dst_e = edge_index[1].astype(jnp.int32)
    msgs = jnp.take(x, src_e, axis=0)
    agg = jax.ops.segment_sum(msgs, dst_e, num_segments=N_NODES)
    deg = jax.ops.segment_sum(jnp.ones(src_e.shape, jnp.float32), dst_e,
                              num_segments=N_NODES)
    mean_agg = agg / jnp.clip(deg, 1.0, None)[:, None]
    h = _dense_stage(x, mean_agg, W_self, W_neigh)
    src32 = src.astype(jnp.int32)
    dst32 = dst.astype(jnp.int32)
    h_src = jnp.take(h, src32, axis=0)
    h_dst = jnp.take(h, dst32, axis=0)
    score = (mu + (h_src * h_dst).sum(axis=1)
             + jnp.take(node_biases, src32 + 1) + jnp.take(node_biases, dst32 + 1))
    return score
